# baseline (speedup 1.0000x reference)
import functools

import jax
import jax.numpy as jnp
from jax import lax
from jax.experimental import pallas as pl
from jax.experimental.pallas import tpu as pltpu
from jax.experimental.pallas import tpu_sc as plsc

N = 10000
E = 10000
INC = 320000
D = 128
NC, NS = 2, 16
NW = NC * NS
K = 80
NCH = 128
PER_W = K * NCH
INC_P = NW * PER_W
NP = 10240
ROWS_PT = NP // NS
ZR = 128

_mesh = plsc.VectorSubcoreMesh(core_axis_name="c", subcore_axis_name="s")


@functools.partial(
    pl.kernel,
    out_type=jax.ShapeDtypeStruct((NC, NP, D), jnp.float32),
    mesh=_mesh,
    scratch_types=[
        pltpu.VMEM((K,), jnp.int32),
        pltpu.VMEM((K,), jnp.int32),
        pltpu.VMEM((K,), jnp.int32),
        pltpu.VMEM((K,), jnp.int32),
        pltpu.VMEM((K, D), jnp.float32),
        pltpu.VMEM((K, D), jnp.float32),
        pltpu.VMEM((ZR, D), jnp.float32),
        pltpu.VMEM_SHARED((NP, D), jnp.float32),
        pltpu.SemaphoreType.DMA,
        pltpu.SemaphoreType.DMA,
    ],
)
def _sc_pass(src_hbm, gidx_hbm, sidx_hbm, zd_hbm, acc_out,
             gk0_v, gk1_v, sk0_v, sk1_v, rows0_v, rows1_v, zd_v, acc_sh,
             semg0, semg1):
    cid = lax.axis_index("c")
    sid = lax.axis_index("s")
    wid = cid * NS + sid

    pltpu.sync_copy(zd_hbm, zd_v)

    base = sid * ROWS_PT
    for r in range(ROWS_PT // ZR):
        pltpu.sync_copy(zd_v, acc_sh.at[pl.ds(base + r * ZR, ZR)])

    base_i = wid * PER_W
    pltpu.sync_copy(gidx_hbm.at[pl.ds(base_i, K)], gk0_v)
    pltpu.sync_copy(sidx_hbm.at[pl.ds(base_i, K)], sk0_v)
    pltpu.sync_copy(gidx_hbm.at[pl.ds(base_i + K, K)], gk1_v)
    pltpu.sync_copy(sidx_hbm.at[pl.ds(base_i + K, K)], sk1_v)

    plsc.subcore_barrier()

    pltpu.async_copy(src_hbm.at[gk0_v], rows0_v, semg0)
    pltpu.async_copy(src_hbm.at[gk1_v], rows1_v, semg1)

    def pair(p, c):
        j0 = 2 * p

        pltpu.make_async_copy(src_hbm.at[gk0_v], rows0_v, semg0).wait()
        pltpu.sync_copy(rows0_v, acc_sh.at[sk0_v], add=True)

        @pl.when(j0 + 2 < NCH)
        def _():
            pltpu.sync_copy(gidx_hbm.at[pl.ds(base_i + (j0 + 2) * K, K)],
                            gk0_v)
            pltpu.sync_copy(sidx_hbm.at[pl.ds(base_i + (j0 + 2) * K, K)],
                            sk0_v)
            pltpu.async_copy(src_hbm.at[gk0_v], rows0_v, semg0)

        pltpu.make_async_copy(src_hbm.at[gk1_v], rows1_v, semg1).wait()
        pltpu.sync_copy(rows1_v, acc_sh.at[sk1_v], add=True)

        @pl.when(j0 + 3 < NCH)
        def _():
            pltpu.sync_copy(gidx_hbm.at[pl.ds(base_i + (j0 + 3) * K, K)],
                            gk1_v)
            pltpu.sync_copy(sidx_hbm.at[pl.ds(base_i + (j0 + 3) * K, K)],
                            sk1_v)
            pltpu.async_copy(src_hbm.at[gk1_v], rows1_v, semg1)

        return c
    lax.fori_loop(0, NCH // 2, pair, 0)

    plsc.subcore_barrier()

    pltpu.sync_copy(acc_sh.at[pl.ds(base, ROWS_PT)],
                    acc_out.at[cid, pl.ds(base, ROWS_PT)])


_BLK = 1024


def _tc_combine1(a0, a1, c0, c1, W):
    def body(a0_r, a1_r, c0_r, c1_r, w_r, o_r):
        s = a0_r[...] + a1_r[...]
        y = jnp.dot(s, w_r[...], preferred_element_type=jnp.float32)
        cnt = jnp.sum(c0_r[...] + c1_r[...], axis=1, keepdims=True) * (1.0 / D)
        inv = jnp.where(cnt > 0, 1.0 / cnt, 0.0)
        o_r[...] = inv * y

    return pl.pallas_call(
        body,
        grid=(NP // _BLK,),
        in_specs=[
            pl.BlockSpec((_BLK, D), lambda i: (i, 0)),
            pl.BlockSpec((_BLK, D), lambda i: (i, 0)),
            pl.BlockSpec((_BLK, D), lambda i: (i, 0)),
            pl.BlockSpec((_BLK, D), lambda i: (i, 0)),
            pl.BlockSpec((D, D), lambda i: (0, 0)),
        ],
        out_specs=pl.BlockSpec((_BLK, D), lambda i: (i, 0)),
        out_shape=jax.ShapeDtypeStruct((NP, D), jnp.float32),
    )(a0, a1, c0, c1, W)


def _tc_combine2(q0, q1, c0, c1, b2d):
    def body(q0_r, q1_r, c0_r, c1_r, b_r, o_r):
        s = q0_r[...] + q1_r[...]
        cnt = jnp.sum(c0_r[...] + c1_r[...], axis=1, keepdims=True) * (1.0 / D)
        inv = jnp.where(cnt > 0, 1.0 / cnt, 0.0)
        o_r[...] = inv * s + b_r[...]

    return pl.pallas_call(
        body,
        grid=(NP // _BLK,),
        in_specs=[
            pl.BlockSpec((_BLK, D), lambda i: (i, 0)),
            pl.BlockSpec((_BLK, D), lambda i: (i, 0)),
            pl.BlockSpec((_BLK, D), lambda i: (i, 0)),
            pl.BlockSpec((_BLK, D), lambda i: (i, 0)),
            pl.BlockSpec((1, D), lambda i: (0, 0)),
        ],
        out_specs=pl.BlockSpec((_BLK, D), lambda i: (i, 0)),
        out_shape=jax.ShapeDtypeStruct((NP, D), jnp.float32),
    )(q0, q1, c0, c1, b2d)


def kernel(x, hyperedge_index, W, b):
    hi = hyperedge_index.astype(jnp.int32)
    pad = jnp.full((INC_P - INC,), NP - 1, jnp.int32)
    nidx = jnp.concatenate([hi[0].reshape(-1), pad])
    eidx = jnp.concatenate([hi[1].reshape(-1), pad])

    xp = jnp.concatenate(
        [x, jnp.zeros((NP - N, D), jnp.float32)], axis=0)

    zdrows = jnp.zeros((ZR, D), jnp.float32)
    ones_tab = jnp.ones((NP, D), jnp.float32)

    cnte = _sc_pass(ones_tab, nidx, eidx, zdrows)
    cntn = _sc_pass(ones_tab, eidx, nidx, zdrows)
    acc = _sc_pass(xp, nidx, eidx, zdrows)
    oute = _tc_combine1(acc[0], acc[1], cnte[0], cnte[1], W)
    q = _sc_pass(oute, eidx, nidx, zdrows)
    out = _tc_combine2(q[0], q[1], cntn[0], cntn[1], b.reshape(1, D))
    return out[:N]


# speedup vs baseline: 2.5332x; 2.5332x over previous
import functools

import jax
import jax.numpy as jnp
from jax import lax
from jax.experimental import pallas as pl
from jax.experimental.pallas import tpu as pltpu
from jax.experimental.pallas import tpu_sc as plsc

N = 10000
E = 10000
INC = 320000
D = 128
NC, NS = 2, 16
NW = NC * NS
K = 120
NCH = 84
PER_W = K * NCH
INC_P = NW * PER_W
NP = 10240
ROWS_PT = NP // NS
ZR = 128

_mesh = plsc.VectorSubcoreMesh(core_axis_name="c", subcore_axis_name="s")


@functools.partial(
    pl.kernel,
    out_type=jax.ShapeDtypeStruct((NC, NP, D), jnp.float32),
    mesh=_mesh,
    scratch_types=[
        pltpu.VMEM((K,), jnp.int32),
        pltpu.VMEM((K,), jnp.int32),
        pltpu.VMEM((K,), jnp.int32),
        pltpu.VMEM((K,), jnp.int32),
        pltpu.VMEM((K, D), jnp.float32),
        pltpu.VMEM((K, D), jnp.float32),
        pltpu.VMEM((ZR, D), jnp.float32),
        pltpu.VMEM_SHARED((NP, D), jnp.float32),
        pltpu.SemaphoreType.DMA,
        pltpu.SemaphoreType.DMA,
    ],
)
def _sc_pass(src_hbm, gidx_hbm, sidx_hbm, zd_hbm, acc_out,
             gk0_v, gk1_v, sk0_v, sk1_v, rows0_v, rows1_v, zd_v, acc_sh,
             semg0, semg1):
    cid = lax.axis_index("c")
    sid = lax.axis_index("s")
    wid = cid * NS + sid

    pltpu.sync_copy(zd_hbm, zd_v)

    base = sid * ROWS_PT
    for r in range(ROWS_PT // ZR):
        pltpu.sync_copy(zd_v, acc_sh.at[pl.ds(base + r * ZR, ZR)])

    base_i = wid * PER_W
    pltpu.sync_copy(gidx_hbm.at[pl.ds(base_i, K)], gk0_v)
    pltpu.sync_copy(sidx_hbm.at[pl.ds(base_i, K)], sk0_v)
    pltpu.sync_copy(gidx_hbm.at[pl.ds(base_i + K, K)], gk1_v)
    pltpu.sync_copy(sidx_hbm.at[pl.ds(base_i + K, K)], sk1_v)

    plsc.subcore_barrier()

    pltpu.async_copy(src_hbm.at[gk0_v], rows0_v, semg0)
    pltpu.async_copy(src_hbm.at[gk1_v], rows1_v, semg1)

    def pair(p, c):
        j0 = 2 * p

        pltpu.make_async_copy(src_hbm.at[gk0_v], rows0_v, semg0).wait()
        pltpu.sync_copy(rows0_v, acc_sh.at[sk0_v], add=True)

        @pl.when(j0 + 2 < NCH)
        def _():
            pltpu.sync_copy(gidx_hbm.at[pl.ds(base_i + (j0 + 2) * K, K)],
                            gk0_v)
            pltpu.sync_copy(sidx_hbm.at[pl.ds(base_i + (j0 + 2) * K, K)],
                            sk0_v)
            pltpu.async_copy(src_hbm.at[gk0_v], rows0_v, semg0)

        pltpu.make_async_copy(src_hbm.at[gk1_v], rows1_v, semg1).wait()
        pltpu.sync_copy(rows1_v, acc_sh.at[sk1_v], add=True)

        @pl.when(j0 + 3 < NCH)
        def _():
            pltpu.sync_copy(gidx_hbm.at[pl.ds(base_i + (j0 + 3) * K, K)],
                            gk1_v)
            pltpu.sync_copy(sidx_hbm.at[pl.ds(base_i + (j0 + 3) * K, K)],
                            sk1_v)
            pltpu.async_copy(src_hbm.at[gk1_v], rows1_v, semg1)

        return c
    lax.fori_loop(0, NCH // 2, pair, 0)

    plsc.subcore_barrier()

    pltpu.sync_copy(acc_sh.at[pl.ds(base, ROWS_PT)],
                    acc_out.at[cid, pl.ds(base, ROWS_PT)])


_BLK = 1024


def _tc_combine1(a0, a1, c0, c1, W):
    def body(a0_r, a1_r, c0_r, c1_r, w_r, o_r):
        s = a0_r[...] + a1_r[...]
        y = jnp.dot(s, w_r[...], preferred_element_type=jnp.float32)
        cnt = jnp.sum(c0_r[...] + c1_r[...], axis=1, keepdims=True) * (1.0 / D)
        inv = jnp.where(cnt > 0, 1.0 / cnt, 0.0)
        o_r[...] = inv * y

    return pl.pallas_call(
        body,
        grid=(NP // _BLK,),
        in_specs=[
            pl.BlockSpec((_BLK, D), lambda i: (i, 0)),
            pl.BlockSpec((_BLK, D), lambda i: (i, 0)),
            pl.BlockSpec((_BLK, D), lambda i: (i, 0)),
            pl.BlockSpec((_BLK, D), lambda i: (i, 0)),
            pl.BlockSpec((D, D), lambda i: (0, 0)),
        ],
        out_specs=pl.BlockSpec((_BLK, D), lambda i: (i, 0)),
        out_shape=jax.ShapeDtypeStruct((NP, D), jnp.float32),
    )(a0, a1, c0, c1, W)


def _tc_combine2(q0, q1, c0, c1, b2d):
    def body(q0_r, q1_r, c0_r, c1_r, b_r, o_r):
        s = q0_r[...] + q1_r[...]
        cnt = jnp.sum(c0_r[...] + c1_r[...], axis=1, keepdims=True) * (1.0 / D)
        inv = jnp.where(cnt > 0, 1.0 / cnt, 0.0)
        o_r[...] = inv * s + b_r[...]

    return pl.pallas_call(
        body,
        grid=(NP // _BLK,),
        in_specs=[
            pl.BlockSpec((_BLK, D), lambda i: (i, 0)),
            pl.BlockSpec((_BLK, D), lambda i: (i, 0)),
            pl.BlockSpec((_BLK, D), lambda i: (i, 0)),
            pl.BlockSpec((_BLK, D), lambda i: (i, 0)),
            pl.BlockSpec((1, D), lambda i: (0, 0)),
        ],
        out_specs=pl.BlockSpec((_BLK, D), lambda i: (i, 0)),
        out_shape=jax.ShapeDtypeStruct((NP, D), jnp.float32),
    )(q0, q1, c0, c1, b2d)


def kernel(x, hyperedge_index, W, b):
    hi = hyperedge_index.astype(jnp.int32)
    pad = N + jax.lax.iota(jnp.int32, INC_P - INC) % (NP - N)
    nidx = jnp.concatenate([hi[0].reshape(-1), pad])
    eidx = jnp.concatenate([hi[1].reshape(-1), pad])

    xp = jnp.concatenate(
        [x, jnp.zeros((NP - N, D), jnp.float32)], axis=0)

    zdrows = jnp.zeros((ZR, D), jnp.float32)
    ones_tab = jnp.ones((NP, D), jnp.float32)

    cnte = _sc_pass(ones_tab, nidx, eidx, zdrows)
    cntn = _sc_pass(ones_tab, eidx, nidx, zdrows)
    acc = _sc_pass(xp, nidx, eidx, zdrows)
    oute = _tc_combine1(acc[0], acc[1], cnte[0], cnte[1], W)
    q = _sc_pass(oute, eidx, nidx, zdrows)
    out = _tc_combine2(q[0], q[1], cntn[0], cntn[1], b.reshape(1, D))
    return out[:N]


# speedup vs baseline: 2.5365x; 1.0013x over previous
import functools

import jax
import jax.numpy as jnp
from jax import lax
from jax.experimental import pallas as pl
from jax.experimental.pallas import tpu as pltpu
from jax.experimental.pallas import tpu_sc as plsc

N = 10000
E = 10000
INC = 320000
D = 128
NC, NS = 2, 16
NW = NC * NS
K = 120
NCH = 84
PER_W = K * NCH
INC_P = NW * PER_W
NP = 10240
ROWS_PT = NP // NS
ZR = 128

_mesh = plsc.VectorSubcoreMesh(core_axis_name="c", subcore_axis_name="s")


@functools.partial(
    pl.kernel,
    out_type=jax.ShapeDtypeStruct((NC, NP, D), jnp.float32),
    mesh=_mesh,
    scratch_types=[
        pltpu.VMEM((K,), jnp.int32),
        pltpu.VMEM((K,), jnp.int32),
        pltpu.VMEM((K,), jnp.int32),
        pltpu.VMEM((K,), jnp.int32),
        pltpu.VMEM((K, D), jnp.float32),
        pltpu.VMEM((K, D), jnp.float32),
        pltpu.VMEM((ZR, D), jnp.float32),
        pltpu.VMEM_SHARED((NP, D), jnp.float32),
        pltpu.SemaphoreType.DMA,
        pltpu.SemaphoreType.DMA,
    ],
)
def _sc_pass(src_hbm, gidx_hbm, sidx_hbm, zd_hbm, acc_out,
             gk0_v, gk1_v, sk0_v, sk1_v, rows0_v, rows1_v, zd_v, acc_sh,
             semg0, semg1):
    cid = lax.axis_index("c")
    sid = lax.axis_index("s")
    wid = cid * NS + sid

    pltpu.sync_copy(zd_hbm, zd_v)

    base = sid * ROWS_PT
    for r in range(ROWS_PT // ZR):
        pltpu.sync_copy(zd_v, acc_sh.at[pl.ds(base + r * ZR, ZR)])

    base_i = wid * PER_W
    pltpu.sync_copy(gidx_hbm.at[pl.ds(base_i, K)], gk0_v)
    pltpu.sync_copy(sidx_hbm.at[pl.ds(base_i, K)], sk0_v)
    pltpu.sync_copy(gidx_hbm.at[pl.ds(base_i + K, K)], gk1_v)
    pltpu.sync_copy(sidx_hbm.at[pl.ds(base_i + K, K)], sk1_v)

    plsc.subcore_barrier()

    pltpu.async_copy(src_hbm.at[gk0_v], rows0_v, semg0)
    pltpu.async_copy(src_hbm.at[gk1_v], rows1_v, semg1)

    def pair(p, c):
        j0 = 2 * p

        pltpu.make_async_copy(src_hbm.at[gk0_v], rows0_v, semg0).wait()
        pltpu.sync_copy(rows0_v, acc_sh.at[sk0_v], add=True)

        @pl.when(j0 + 2 < NCH)
        def _():
            pltpu.sync_copy(gidx_hbm.at[pl.ds(base_i + (j0 + 2) * K, K)],
                            gk0_v)
            pltpu.sync_copy(sidx_hbm.at[pl.ds(base_i + (j0 + 2) * K, K)],
                            sk0_v)
            pltpu.async_copy(src_hbm.at[gk0_v], rows0_v, semg0)

        pltpu.make_async_copy(src_hbm.at[gk1_v], rows1_v, semg1).wait()
        pltpu.sync_copy(rows1_v, acc_sh.at[sk1_v], add=True)

        @pl.when(j0 + 3 < NCH)
        def _():
            pltpu.sync_copy(gidx_hbm.at[pl.ds(base_i + (j0 + 3) * K, K)],
                            gk1_v)
            pltpu.sync_copy(sidx_hbm.at[pl.ds(base_i + (j0 + 3) * K, K)],
                            sk1_v)
            pltpu.async_copy(src_hbm.at[gk1_v], rows1_v, semg1)

        return c
    lax.fori_loop(0, NCH // 2, pair, 0)

    plsc.subcore_barrier()

    pltpu.sync_copy(acc_sh.at[pl.ds(base, ROWS_PT)],
                    acc_out.at[cid, pl.ds(base, ROWS_PT)])


_BLK = 1024


def _tc_combine1(a0, a1, c0, c1, W):
    def body(a0_r, a1_r, c0_r, c1_r, w_r, o_r):
        s = a0_r[...] + a1_r[...]
        y = jnp.dot(s, w_r[...], preferred_element_type=jnp.float32)
        cnt = jnp.sum(c0_r[...] + c1_r[...], axis=1, keepdims=True) * (1.0 / D)
        inv = jnp.where(cnt > 0, 1.0 / cnt, 0.0)
        o_r[...] = inv * y

    return pl.pallas_call(
        body,
        grid=(NP // _BLK,),
        in_specs=[
            pl.BlockSpec((_BLK, D), lambda i: (i, 0)),
            pl.BlockSpec((_BLK, D), lambda i: (i, 0)),
            pl.BlockSpec((_BLK, D), lambda i: (i, 0)),
            pl.BlockSpec((_BLK, D), lambda i: (i, 0)),
            pl.BlockSpec((D, D), lambda i: (0, 0)),
        ],
        out_specs=pl.BlockSpec((_BLK, D), lambda i: (i, 0)),
        out_shape=jax.ShapeDtypeStruct((NP, D), jnp.float32),
    )(a0, a1, c0, c1, W)


def _tc_combine2(q0, q1, c0, c1, b2d):
    def body(q0_r, q1_r, c0_r, c1_r, b_r, o_r):
        s = q0_r[...] + q1_r[...]
        cnt = jnp.sum(c0_r[...] + c1_r[...], axis=1, keepdims=True) * (1.0 / D)
        inv = jnp.where(cnt > 0, 1.0 / cnt, 0.0)
        o_r[...] = inv * s + b_r[...]

    return pl.pallas_call(
        body,
        grid=(NP // _BLK,),
        in_specs=[
            pl.BlockSpec((_BLK, D), lambda i: (i, 0)),
            pl.BlockSpec((_BLK, D), lambda i: (i, 0)),
            pl.BlockSpec((_BLK, D), lambda i: (i, 0)),
            pl.BlockSpec((_BLK, D), lambda i: (i, 0)),
            pl.BlockSpec((1, D), lambda i: (0, 0)),
        ],
        out_specs=pl.BlockSpec((_BLK, D), lambda i: (i, 0)),
        out_shape=jax.ShapeDtypeStruct((NP, D), jnp.float32),
    )(q0, q1, c0, c1, b2d)


def kernel(x, hyperedge_index, W, b):
    hi = hyperedge_index.astype(jnp.int32)
    pad = N + jax.lax.iota(jnp.int32, INC_P - INC) % (NP - N)
    nidx = jnp.concatenate([hi[0].reshape(-1), pad])
    eidx = jnp.concatenate([hi[1].reshape(-1), pad])

    xp = jnp.concatenate(
        [x, jnp.zeros((NP - N, D), jnp.float32)], axis=0)

    zdrows = jnp.zeros((ZR, D), jnp.float32)
    ones_tab = jnp.ones((NP, D), jnp.float32)

    lin_idx = jax.lax.iota(jnp.int32, INC_P) % N
    cnte = _sc_pass(ones_tab, lin_idx, eidx, zdrows)
    cntn = _sc_pass(ones_tab, lin_idx, nidx, zdrows)
    acc = _sc_pass(xp, nidx, eidx, zdrows)
    oute = _tc_combine1(acc[0], acc[1], cnte[0], cnte[1], W)
    q = _sc_pass(oute, eidx, nidx, zdrows)
    out = _tc_combine2(q[0], q[1], cntn[0], cntn[1], b.reshape(1, D))
    return out[:N]
